# baseline (speedup 1.0000x reference)
import functools

import jax
import jax.numpy as jnp
from jax import lax
from jax.experimental import pallas as pl
from jax.experimental.pallas import tpu as pltpu
from jax.experimental.pallas import tpu_sc as plsc

_N = 10000
_D = 128
_E = 320000
_NG = 2
_B = 16384
_L = 16
_NSUB = 16
_NCORES = 2

_EPT = _E // _NSUB
_CKA = 800
_NBIG = _EPT // _CKA
_SUB = 40
_NSC = _CKA // _SUB
_ROT = 640

_mesh = plsc.VectorSubcoreMesh(
    core_axis_name="c", subcore_axis_name="s",
    num_cores=_NCORES, num_subcores=_NSUB)


def _lrelu(x):
    return jnp.maximum(x, 0.2 * x)


def _tc1_body(s_ref, w1_ref, a_ref, h1_ref, asan_ref):
    S = s_ref[0]
    H1 = jnp.dot(S, w1_ref[0], preferred_element_type=jnp.float32)
    h1_ref[0] = H1
    asan_ref[0, 0, :] = jnp.sum(H1 * a_ref[0, 0, :], axis=1)
    asan_ref[0, 1, :] = jnp.sum(H1 * a_ref[0, 1, :], axis=1)


def _tc1(S, W1, A):
    return pl.pallas_call(
        _tc1_body,
        grid=(2,),
        in_specs=[
            pl.BlockSpec((1, _N, _D), lambda s: (s, 0, 0)),
            pl.BlockSpec((1, _D, _D), lambda s: (s, 0, 0)),
            pl.BlockSpec((1, 2, _D), lambda s: (s, 0, 0)),
        ],
        out_specs=[
            pl.BlockSpec((1, _N, _D), lambda s: (s, 0, 0)),
            pl.BlockSpec((1, 2, _N), lambda s: (s, 0, 0)),
        ],
        out_shape=[
            jax.ShapeDtypeStruct((2, _N, _D), jnp.float32),
            jax.ShapeDtypeStruct((2, 2, _N), jnp.float32),
        ],
    )(S, W1, A)


@functools.partial(
    pl.kernel,
    out_type=[jax.ShapeDtypeStruct((2, _N, _D), jnp.float32),
              jax.ShapeDtypeStruct((2, _N, _D), jnp.float32)],
    mesh=_mesh,
    scratch_types=[
        pltpu.VMEM((_N,), jnp.float32),
        pltpu.VMEM((_N,), jnp.float32),
        pltpu.VMEM((_ROT + _L,), jnp.float32),
        pltpu.VMEM((_CKA,), jnp.int32),
        pltpu.VMEM((_CKA,), jnp.int32),
        pltpu.VMEM((_NSC, _SUB), jnp.int32),
        pltpu.VMEM((_CKA + _L,), jnp.float32),
        pltpu.VMEM((4, _SUB, _D), jnp.float32),
        pltpu.VMEM((2 * _NG, _L), jnp.float32),
        pltpu.VMEM_SHARED((_N, _D), jnp.float32),
        pltpu.VMEM_SHARED((_N,), jnp.float32),
        pltpu.VMEM_SHARED((_N,), jnp.float32),
        pltpu.SemaphoreType.DMA,
        pltpu.SemaphoreType.DMA,
        pltpu.SemaphoreType.DMA,
    ],
    compiler_params=pltpu.CompilerParams(needs_layout_passes=False),
)
def _sc_main(srcf, dstf, dst5, h1cat, asan, om, h2out0, h2out1,
             as_v, an_v, invw, src_a, dst_a1, dst_a2, w_a, rows_v,
             om_v, h2_sp, den0_sp, den1_sp, sem_g, sem_s, sem_r):
    core = lax.axis_index("c")
    sid = lax.axis_index("s")
    zero16 = jnp.zeros((_L,), jnp.float32)
    r0 = sid * _ROT

    def _zrow(j, _):
        for seg in range(_D // _L):
            rows_v[0, j, pl.ds(seg * _L, _L)] = zero16
            rows_v[1, j, pl.ds(seg * _L, _L)] = zero16
        return _
    lax.fori_loop(0, _SUB, _zrow, None)

    def _zw(i, _):
        w_a[pl.ds(i * _L, _L)] = zero16
        return _
    lax.fori_loop(0, (_CKA + _L) // _L, _zw, None)

    def _zero_h2():
        def _zh(k, _):
            start = r0 + k * _SUB

            @pl.when(start < _N)
            def _do():
                pltpu.sync_copy(rows_v.at[0], h2_sp.at[pl.ds(start, _SUB)])
            return _
        lax.fori_loop(0, _ROT // _SUB, _zh, None)

    _zero_h2()

    @pl.when(sid < 10)
    def _zero_den():
        for den in (den0_sp, den1_sp):
            for off, n in ((0, 400), (400, 400), (800, 200)):
                pltpu.sync_copy(w_a.at[pl.ds(0, n)],
                                den.at[pl.ds(sid * 1000 + off, n)])

    pltpu.sync_copy(asan.at[pl.ds(core * 2 * _N, _N)], as_v)
    pltpu.sync_copy(asan.at[pl.ds(core * 2 * _N + _N, _N)], an_v)
    pltpu.sync_copy(om, om_v)

    plsc.subcore_barrier()

    for g, den_g, h2out_g in ((0, den0_sp, h2out0), (1, den1_sp, h2out1)):

        def _big(bc, _, g=g, den_g=den_g):
            gg = core * _NG + g
            goff = gg * _E + sid * _EPT + bc * _CKA
            d1 = pltpu.async_copy(srcf.at[pl.ds(goff, _CKA)], src_a, sem_s)
            d2 = pltpu.async_copy(dstf.at[pl.ds(goff, _CKA)], dst_a1, sem_s)
            d3 = pltpu.async_copy(dst5.at[gg, sid, bc], dst_a2, sem_s)
            d1.wait(); d2.wait(); d3.wait()
            coff = core * _N

            def _wv(k, __):
                s16 = src_a[pl.ds(k * _L, _L)]
                d16 = dst_a1[pl.ds(k * _L, _L)]
                sc = plsc.load_gather(as_v, [d16]) + plsc.load_gather(an_v, [s16])
                w_a[pl.ds(k * _L, _L)] = jnp.exp(_lrelu(sc))
                src_a[pl.ds(k * _L, _L)] = s16 + coff
                return __
            lax.fori_loop(0, _CKA // _L, _wv, None)

            dds = [pltpu.async_copy(w_a.at[pl.ds(s * _SUB, _SUB)],
                                    den_g.at[dst_a2.at[s]], sem_s, add=True)
                   for s in range(_NSC)]

            for kk in range(3):
                pltpu.async_copy(h1cat.at[src_a.at[pl.ds(kk * _SUB, _SUB)]],
                                 rows_v.at[kk], sem_g)

            def _sub(k, __, den_g=den_g):
                sl = lax.rem(k, 4)

                @pl.when(k >= 1)
                def _drain_prev():
                    pltpu.make_async_copy(rows_v.at[sl],
                                          h2_sp.at[dst_a2.at[k]],
                                          sem_r).wait()

                pltpu.make_async_copy(h1cat.at[src_a.at[pl.ds(0, _SUB)]],
                                      rows_v.at[sl], sem_g).wait()

                @pl.when(k + 3 < _NSC)
                def _issue_next():
                    pltpu.async_copy(
                        h1cat.at[src_a.at[pl.ds((k + 3) * _SUB, _SUB)]],
                        rows_v.at[lax.rem(k + 3, 4)], sem_g)

                base = k * _SUB
                for jj in range(_SUB):
                    av = plsc.load_gather(
                        w_a, [jnp.full((_L,), base + jj, jnp.int32)])
                    for seg in range(_D // _L):
                        rows_v[sl, jj, pl.ds(seg * _L, _L)] = (
                            rows_v[sl, jj, pl.ds(seg * _L, _L)] * av)

                pltpu.async_copy(rows_v.at[sl], h2_sp.at[dst_a2.at[k]],
                                 sem_r, add=True)
                return __
            lax.fori_loop(0, _NSC, _sub, None)
            pltpu.make_async_copy(rows_v.at[0], h2_sp.at[dst_a2.at[0]],
                                  sem_r).wait()
            for dd in dds:
                dd.wait()
            return _
        lax.fori_loop(0, _NBIG, _big, None)

        plsc.subcore_barrier()

        om16 = om_v[core * _NG + g]

        def _stg(k, _, den_g=den_g, om16=om16):
            start = r0 + k * _SUB

            @pl.when(start < _N)
            def _do():
                pltpu.sync_copy(den_g.at[pl.ds(start, _SUB)],
                                invw.at[pl.ds(k * _SUB, _SUB)])
            return _
        lax.fori_loop(0, _ROT // _SUB, _stg, None)

        def _inv(i, _, om16=om16):
            d = invw[pl.ds(i * _L, _L)]
            invw[pl.ds(i * _L, _L)] = om16 / (d + 1e-16)
            return _
        lax.fori_loop(0, _ROT // _L, _inv, None)

        def _zr1(j, _):
            for seg in range(_D // _L):
                rows_v[1, j, pl.ds(seg * _L, _L)] = zero16
            return _
        lax.fori_loop(0, _SUB, _zr1, None)

        def _wout(k, _, h2out_g=h2out_g):
            start = r0 + k * _SUB

            @pl.when(start < _N)
            def _do():
                pltpu.sync_copy(h2_sp.at[pl.ds(start, _SUB)], rows_v.at[0])
                base = k * _SUB
                for grp in range((_SUB + _L - 1) // _L):
                    a16 = invw[pl.ds(base + grp * _L, _L)]
                    for jl in range(min(_L, _SUB - grp * _L)):
                        jj = grp * _L + jl
                        av = jnp.broadcast_to(a16[jl], (_L,))
                        for seg in range(_D // _L):
                            rows_v[0, jj, pl.ds(seg * _L, _L)] = (
                                rows_v[0, jj, pl.ds(seg * _L, _L)] * av)
                pltpu.sync_copy(rows_v.at[0],
                                h2out_g.at[core, pl.ds(start, _SUB)])
                pltpu.sync_copy(rows_v.at[1], h2_sp.at[pl.ds(start, _SUB)])
            return _
        lax.fori_loop(0, _ROT // _SUB, _wout, None)

        plsc.subcore_barrier()


def _elu(x):
    return jnp.where(x > 0, x, jnp.exp(x) - 1.0)


def _tc2_body(h2a_ref, h2b_ref, s_ref, w2_ref, ws2_ref, b2_ref, w3_ref,
              h4_ref, u_ref):
    H2 = h2a_ref[0] + h2b_ref[0]
    H3 = _elu(jnp.dot(H2, w2_ref[0], preferred_element_type=jnp.float32)
              + jnp.dot(s_ref[0], ws2_ref[0], preferred_element_type=jnp.float32)
              + b2_ref[0, 0, :])
    u_ref[0] = _elu(jnp.dot(H3, w3_ref[0],
                            preferred_element_type=jnp.float32)) + h4_ref[0]


def _tc2(H2a, H2b, S, W2, Ws2, b2, W3, H4):
    full = lambda s: (s, 0, 0)
    return pl.pallas_call(
        _tc2_body,
        grid=(2,),
        in_specs=[
            pl.BlockSpec((1, _N, _D), full),
            pl.BlockSpec((1, _N, _D), full),
            pl.BlockSpec((1, _N, _D), full),
            pl.BlockSpec((1, _D, _D), full),
            pl.BlockSpec((1, _D, _D), full),
            pl.BlockSpec((1, 1, _D), full),
            pl.BlockSpec((1, _D, _D), full),
            pl.BlockSpec((1, _N, _D), full),
        ],
        out_specs=pl.BlockSpec((1, _N, _D), full),
        out_shape=jax.ShapeDtypeStruct((2, _N, _D), jnp.float32),
    )(H2a, H2b, S, W2, Ws2, b2, W3, H4)


_BPT = _B // (_NCORES * _NSUB)
_FCK = 128

@functools.partial(
    pl.kernel,
    out_type=jax.ShapeDtypeStruct((_B,), jnp.float32),
    mesh=_mesh,
    scratch_types=[
        pltpu.VMEM((_BPT,), jnp.int32),
        pltpu.VMEM((_BPT,), jnp.int32),
        pltpu.VMEM((_FCK, _D), jnp.float32),
        pltpu.VMEM((_FCK, _D), jnp.float32),
        pltpu.VMEM((_N,), jnp.float32),
        pltpu.VMEM((_N,), jnp.float32),
        pltpu.VMEM((_L,), jnp.float32),
        pltpu.VMEM((_L, _L), jnp.float32),
        pltpu.VMEM((_BPT,), jnp.float32),
        pltpu.SemaphoreType.DMA,
    ],
    compiler_params=pltpu.CompilerParams(needs_layout_passes=False),
)
def _sc_final(uu, ub, uidx, iidx, bub, bx, r_out,
              ui_v, ii_v, urows, brows, buv, bbv, bxv, tbuf, out_v, sem):
    core = lax.axis_index("c")
    sid = lax.axis_index("s")
    wid = core * _NSUB + sid
    base = wid * _BPT

    pltpu.sync_copy(uidx.at[pl.ds(base, _BPT)], ui_v)
    pltpu.sync_copy(iidx.at[pl.ds(base, _BPT)], ii_v)
    pltpu.sync_copy(bub.at[pl.ds(0, _N)], buv)
    pltpu.sync_copy(bub.at[pl.ds(_N, _N)], bbv)
    pltpu.sync_copy(bx, bxv)
    bx16 = bxv[...]
    iota16 = jnp.arange(_L, dtype=jnp.int32)

    def _chunk(cc, _):
        o = cc * _FCK
        pltpu.async_copy(uu.at[ui_v.at[pl.ds(o, _FCK)]], urows, sem).wait()
        pltpu.async_copy(ub.at[ii_v.at[pl.ds(o, _FCK)]], brows, sem).wait()

        def _group(gi, _g):
            j0 = gi * _L
            for jj in range(_L):
                acc = urows[j0 + jj, pl.ds(0, _L)] * brows[j0 + jj, pl.ds(0, _L)]
                for seg in range(1, _D // _L):
                    acc = acc + (urows[j0 + jj, pl.ds(seg * _L, _L)]
                                 * brows[j0 + jj, pl.ds(seg * _L, _L)])
                plsc.store_scatter(
                    tbuf, [iota16, jnp.full((_L,), jj, jnp.int32)], acc)
            tot = tbuf[0, :]
            for i in range(1, _L):
                tot = tot + tbuf[i, :]
            u16 = ui_v[pl.ds(o + j0, _L)]
            i16 = ii_v[pl.ds(o + j0, _L)]
            raw = (tot + plsc.load_gather(buv, [u16])
                   + plsc.load_gather(bbv, [i16]) + bx16)
            out_v[pl.ds(o + j0, _L)] = 4.0 / (1.0 + jnp.exp(-raw)) + 1.0
            return _g
        lax.fori_loop(0, _FCK // _L, _group, None)
        return _
    lax.fori_loop(0, _BPT // _FCK, _chunk, None)

    pltpu.sync_copy(out_v, r_out.at[pl.ds(base, _BPT)])


def kernel(user_indices, item_indices, S_u, S_b, edge_indices_u,
           edge_indices_b, W1_u, a_self_u, a_nb_u, omega_u, W1_b, a_self_b,
           a_nb_b, omega_b, W_u_2, W_us_2, b_us_2, W_b_2, W_bs_2, b_bs_2,
           W_u_3, W_b_3, H_u_4, H_b_4, b_u_x, b_b_x, b_x):
    S = jnp.stack([S_u, S_b])
    W1 = jnp.stack([W1_u, W1_b])
    A = jnp.stack([jnp.concatenate([a_self_u, a_nb_u], axis=0),
                   jnp.concatenate([a_self_b, a_nb_b], axis=0)])
    H1, asan = _tc1(S, W1, A)

    edges = jnp.concatenate(
        [edge_indices_u, edge_indices_b]).astype(jnp.int32)
    srcf = edges[:, 0, :].reshape(-1)
    dstf = edges[:, 1, :].reshape(-1)
    dst5 = dstf.reshape(2 * _NG, _NSUB, _NBIG, _NSC, _SUB)
    h1cat = H1.reshape(2 * _N, _D)
    om = jnp.broadcast_to(
        jnp.concatenate([omega_u, omega_b]).astype(jnp.float32)[:, None],
        (2 * _NG, _L))
    H2a, H2b = _sc_main(srcf, dstf, dst5, h1cat, asan.reshape(-1), om)

    W2 = jnp.stack([W_u_2, W_b_2])
    Ws2 = jnp.stack([W_us_2, W_bs_2])
    b2 = jnp.stack([b_us_2, b_bs_2])[:, None, :]
    W3 = jnp.stack([W_u_3, W_b_3])
    H4 = jnp.stack([H_u_4, H_b_4])
    U = _tc2(H2a, H2b, S, W2, Ws2, b2, W3, H4)

    bub = jnp.concatenate([b_u_x[:, 0], b_b_x[:, 0]]).astype(jnp.float32)
    bx16 = jnp.full((_L,), b_x[0], jnp.float32)
    return _sc_final(U[0], U[1], user_indices.astype(jnp.int32),
                     item_indices.astype(jnp.int32), bub, bx16)


# speedup vs baseline: 1.1789x; 1.1789x over previous
import functools

import jax
import jax.numpy as jnp
from jax import lax
from jax.experimental import pallas as pl
from jax.experimental.pallas import tpu as pltpu
from jax.experimental.pallas import tpu_sc as plsc

_N = 10000
_D = 128
_E = 320000
_NG = 2
_B = 16384
_L = 16
_NSUB = 16
_NCORES = 2

_EPT = _E // _NSUB
_CKA = 800
_NBIG = _EPT // _CKA
_SUB = 40
_NSC = _CKA // _SUB
_ROT = 640

_mesh = plsc.VectorSubcoreMesh(
    core_axis_name="c", subcore_axis_name="s",
    num_cores=_NCORES, num_subcores=_NSUB)


def _lrelu(x):
    return jnp.maximum(x, 0.2 * x)


def _tc1_body(s_ref, w1_ref, a_ref, h1_ref, asan_ref):
    S = s_ref[0]
    H1 = jnp.dot(S, w1_ref[0], preferred_element_type=jnp.float32)
    h1_ref[0] = H1
    asan_ref[0, 0, :] = jnp.sum(H1 * a_ref[0, 0, :], axis=1)
    asan_ref[0, 1, :] = jnp.sum(H1 * a_ref[0, 1, :], axis=1)


def _tc1(S, W1, A):
    return pl.pallas_call(
        _tc1_body,
        grid=(2,),
        in_specs=[
            pl.BlockSpec((1, _N, _D), lambda s: (s, 0, 0)),
            pl.BlockSpec((1, _D, _D), lambda s: (s, 0, 0)),
            pl.BlockSpec((1, 2, _D), lambda s: (s, 0, 0)),
        ],
        out_specs=[
            pl.BlockSpec((1, _N, _D), lambda s: (s, 0, 0)),
            pl.BlockSpec((1, 2, _N), lambda s: (s, 0, 0)),
        ],
        out_shape=[
            jax.ShapeDtypeStruct((2, _N, _D), jnp.float32),
            jax.ShapeDtypeStruct((2, 2, _N), jnp.float32),
        ],
    )(S, W1, A)


@functools.partial(
    pl.kernel,
    out_type=[jax.ShapeDtypeStruct((2, _N, _D), jnp.float32),
              jax.ShapeDtypeStruct((2, _N, _D), jnp.float32)],
    mesh=_mesh,
    scratch_types=[
        pltpu.VMEM((_N,), jnp.float32),
        pltpu.VMEM((_N,), jnp.float32),
        pltpu.VMEM((_ROT + _L,), jnp.float32),
        pltpu.VMEM((_CKA,), jnp.int32),
        pltpu.VMEM((_CKA,), jnp.int32),
        pltpu.VMEM((_NSC, _SUB), jnp.int32),
        pltpu.VMEM((_CKA + _L,), jnp.float32),
        pltpu.VMEM((4, _SUB, _D), jnp.float32),
        pltpu.VMEM((2 * _NG, _L), jnp.float32),
        pltpu.VMEM_SHARED((_N, _D), jnp.float32),
        pltpu.VMEM_SHARED((_N,), jnp.float32),
        pltpu.VMEM_SHARED((_N,), jnp.float32),
        pltpu.SemaphoreType.DMA,
        pltpu.SemaphoreType.DMA,
        pltpu.SemaphoreType.DMA,
    ],
    compiler_params=pltpu.CompilerParams(needs_layout_passes=False),
)
def _sc_main(srcf, dstf, dst5, h1cat, asan, om, h2out0, h2out1,
             as_v, an_v, invw, src_a, dst_a1, dst_a2, w_a, rows_v,
             om_v, h2_sp, den0_sp, den1_sp, sem_g, sem_s, sem_r):
    core = lax.axis_index("c")
    sid = lax.axis_index("s")
    zero16 = jnp.zeros((_L,), jnp.float32)
    r0 = sid * _ROT

    def _zrow(j, _):
        for seg in range(_D // _L):
            rows_v[0, j, pl.ds(seg * _L, _L)] = zero16
            rows_v[1, j, pl.ds(seg * _L, _L)] = zero16
        return _
    lax.fori_loop(0, _SUB, _zrow, None)

    def _zw(i, _):
        w_a[pl.ds(i * _L, _L)] = zero16
        return _
    lax.fori_loop(0, (_CKA + _L) // _L, _zw, None)

    def _zero_h2():
        def _zh(k, _):
            start = r0 + k * _SUB

            @pl.when(start < _N)
            def _do():
                pltpu.sync_copy(rows_v.at[0], h2_sp.at[pl.ds(start, _SUB)])
            return _
        lax.fori_loop(0, _ROT // _SUB, _zh, None)

    _zero_h2()

    @pl.when(sid < 10)
    def _zero_den():
        for den in (den0_sp, den1_sp):
            for off, n in ((0, 400), (400, 400), (800, 200)):
                pltpu.sync_copy(w_a.at[pl.ds(0, n)],
                                den.at[pl.ds(sid * 1000 + off, n)])

    pltpu.sync_copy(asan.at[pl.ds(core * 2 * _N, _N)], as_v)
    pltpu.sync_copy(asan.at[pl.ds(core * 2 * _N + _N, _N)], an_v)
    pltpu.sync_copy(om, om_v)

    plsc.subcore_barrier()

    for g, den_g, h2out_g in ((0, den0_sp, h2out0), (1, den1_sp, h2out1)):

        def _big(bc, _, g=g, den_g=den_g):
            gg = core * _NG + g
            goff = gg * _E + sid * _EPT + bc * _CKA
            d1 = pltpu.async_copy(srcf.at[pl.ds(goff, _CKA)], src_a, sem_s)
            d2 = pltpu.async_copy(dstf.at[pl.ds(goff, _CKA)], dst_a1, sem_s)
            d3 = pltpu.async_copy(dst5.at[gg, sid, bc], dst_a2, sem_s)
            d1.wait(); d2.wait(); d3.wait()
            coff = core * _N

            def _wv(k, __):
                s16 = src_a[pl.ds(k * _L, _L)]
                d16 = dst_a1[pl.ds(k * _L, _L)]
                sc = plsc.load_gather(as_v, [d16]) + plsc.load_gather(an_v, [s16])
                w_a[pl.ds(k * _L, _L)] = jnp.exp(_lrelu(sc))
                src_a[pl.ds(k * _L, _L)] = s16 + coff
                return __
            lax.fori_loop(0, _CKA // _L, _wv, None)

            dds = [pltpu.async_copy(w_a.at[pl.ds(s * _SUB, _SUB)],
                                    den_g.at[dst_a2.at[s]], sem_s, add=True)
                   for s in range(_NSC)]

            for kk in range(3):
                pltpu.async_copy(h1cat.at[src_a.at[pl.ds(kk * _SUB, _SUB)]],
                                 rows_v.at[kk], sem_g)

            def _sub(k, __, den_g=den_g):
                sl = lax.rem(k, 4)

                @pl.when(k >= 1)
                def _drain_prev():
                    pltpu.make_async_copy(rows_v.at[sl],
                                          h2_sp.at[dst_a2.at[k]],
                                          sem_r).wait()

                pltpu.make_async_copy(h1cat.at[src_a.at[pl.ds(0, _SUB)]],
                                      rows_v.at[sl], sem_g).wait()

                @pl.when(k + 3 < _NSC)
                def _issue_next():
                    pltpu.async_copy(
                        h1cat.at[src_a.at[pl.ds((k + 3) * _SUB, _SUB)]],
                        rows_v.at[lax.rem(k + 3, 4)], sem_g)

                base = k * _SUB
                for grp in range((_SUB + _L - 1) // _L):
                    a16 = w_a[pl.ds(base + grp * _L, _L)]
                    for jl in range(min(_L, _SUB - grp * _L)):
                        jj = grp * _L + jl
                        av = jnp.broadcast_to(a16[jl], (_L,))
                        for seg in range(_D // _L):
                            rows_v[sl, jj, pl.ds(seg * _L, _L)] = (
                                rows_v[sl, jj, pl.ds(seg * _L, _L)] * av)

                pltpu.async_copy(rows_v.at[sl], h2_sp.at[dst_a2.at[k]],
                                 sem_r, add=True)
                return __
            lax.fori_loop(0, _NSC, _sub, None)
            pltpu.make_async_copy(rows_v.at[0], h2_sp.at[dst_a2.at[0]],
                                  sem_r).wait()
            for dd in dds:
                dd.wait()
            return _
        lax.fori_loop(0, _NBIG, _big, None)

        plsc.subcore_barrier()

        om16 = om_v[core * _NG + g]

        def _stg(k, _, den_g=den_g, om16=om16):
            start = r0 + k * _SUB

            @pl.when(start < _N)
            def _do():
                pltpu.sync_copy(den_g.at[pl.ds(start, _SUB)],
                                invw.at[pl.ds(k * _SUB, _SUB)])
            return _
        lax.fori_loop(0, _ROT // _SUB, _stg, None)

        def _inv(i, _, om16=om16):
            d = invw[pl.ds(i * _L, _L)]
            invw[pl.ds(i * _L, _L)] = om16 / (d + 1e-16)
            return _
        lax.fori_loop(0, _ROT // _L, _inv, None)

        def _zr1(j, _):
            for seg in range(_D // _L):
                rows_v[1, j, pl.ds(seg * _L, _L)] = zero16
            return _
        lax.fori_loop(0, _SUB, _zr1, None)

        def _wout(k, _, h2out_g=h2out_g):
            start = r0 + k * _SUB

            @pl.when(start < _N)
            def _do():
                pltpu.sync_copy(h2_sp.at[pl.ds(start, _SUB)], rows_v.at[0])
                base = k * _SUB
                for grp in range((_SUB + _L - 1) // _L):
                    a16 = invw[pl.ds(base + grp * _L, _L)]
                    for jl in range(min(_L, _SUB - grp * _L)):
                        jj = grp * _L + jl
                        av = jnp.broadcast_to(a16[jl], (_L,))
                        for seg in range(_D // _L):
                            rows_v[0, jj, pl.ds(seg * _L, _L)] = (
                                rows_v[0, jj, pl.ds(seg * _L, _L)] * av)
                pltpu.sync_copy(rows_v.at[0],
                                h2out_g.at[core, pl.ds(start, _SUB)])
                pltpu.sync_copy(rows_v.at[1], h2_sp.at[pl.ds(start, _SUB)])
            return _
        lax.fori_loop(0, _ROT // _SUB, _wout, None)

        plsc.subcore_barrier()


def _elu(x):
    return jnp.where(x > 0, x, jnp.exp(x) - 1.0)


def _tc2_body(h2a_ref, h2b_ref, s_ref, w2_ref, ws2_ref, b2_ref, w3_ref,
              h4_ref, u_ref):
    H2 = h2a_ref[0] + h2b_ref[0]
    H3 = _elu(jnp.dot(H2, w2_ref[0], preferred_element_type=jnp.float32)
              + jnp.dot(s_ref[0], ws2_ref[0], preferred_element_type=jnp.float32)
              + b2_ref[0, 0, :])
    u_ref[0] = _elu(jnp.dot(H3, w3_ref[0],
                            preferred_element_type=jnp.float32)) + h4_ref[0]


def _tc2(H2a, H2b, S, W2, Ws2, b2, W3, H4):
    full = lambda s: (s, 0, 0)
    return pl.pallas_call(
        _tc2_body,
        grid=(2,),
        in_specs=[
            pl.BlockSpec((1, _N, _D), full),
            pl.BlockSpec((1, _N, _D), full),
            pl.BlockSpec((1, _N, _D), full),
            pl.BlockSpec((1, _D, _D), full),
            pl.BlockSpec((1, _D, _D), full),
            pl.BlockSpec((1, 1, _D), full),
            pl.BlockSpec((1, _D, _D), full),
            pl.BlockSpec((1, _N, _D), full),
        ],
        out_specs=pl.BlockSpec((1, _N, _D), full),
        out_shape=jax.ShapeDtypeStruct((2, _N, _D), jnp.float32),
    )(H2a, H2b, S, W2, Ws2, b2, W3, H4)


_BPT = _B // (_NCORES * _NSUB)
_FCK = 128

@functools.partial(
    pl.kernel,
    out_type=jax.ShapeDtypeStruct((_B,), jnp.float32),
    mesh=_mesh,
    scratch_types=[
        pltpu.VMEM((_BPT,), jnp.int32),
        pltpu.VMEM((_BPT,), jnp.int32),
        pltpu.VMEM((_FCK, _D), jnp.float32),
        pltpu.VMEM((_FCK, _D), jnp.float32),
        pltpu.VMEM((_N,), jnp.float32),
        pltpu.VMEM((_N,), jnp.float32),
        pltpu.VMEM((_L,), jnp.float32),
        pltpu.VMEM((_L, _L), jnp.float32),
        pltpu.VMEM((_BPT,), jnp.float32),
        pltpu.SemaphoreType.DMA,
    ],
    compiler_params=pltpu.CompilerParams(needs_layout_passes=False),
)
def _sc_final(uu, ub, uidx, iidx, bub, bx, r_out,
              ui_v, ii_v, urows, brows, buv, bbv, bxv, tbuf, out_v, sem):
    core = lax.axis_index("c")
    sid = lax.axis_index("s")
    wid = core * _NSUB + sid
    base = wid * _BPT

    pltpu.sync_copy(uidx.at[pl.ds(base, _BPT)], ui_v)
    pltpu.sync_copy(iidx.at[pl.ds(base, _BPT)], ii_v)
    pltpu.sync_copy(bub.at[pl.ds(0, _N)], buv)
    pltpu.sync_copy(bub.at[pl.ds(_N, _N)], bbv)
    pltpu.sync_copy(bx, bxv)
    bx16 = bxv[...]
    iota16 = jnp.arange(_L, dtype=jnp.int32)

    def _chunk(cc, _):
        o = cc * _FCK
        pltpu.async_copy(uu.at[ui_v.at[pl.ds(o, _FCK)]], urows, sem).wait()
        pltpu.async_copy(ub.at[ii_v.at[pl.ds(o, _FCK)]], brows, sem).wait()

        def _group(gi, _g):
            j0 = gi * _L
            for jj in range(_L):
                acc = urows[j0 + jj, pl.ds(0, _L)] * brows[j0 + jj, pl.ds(0, _L)]
                for seg in range(1, _D // _L):
                    acc = acc + (urows[j0 + jj, pl.ds(seg * _L, _L)]
                                 * brows[j0 + jj, pl.ds(seg * _L, _L)])
                plsc.store_scatter(
                    tbuf, [iota16, jnp.full((_L,), jj, jnp.int32)], acc)
            tot = tbuf[0, :]
            for i in range(1, _L):
                tot = tot + tbuf[i, :]
            u16 = ui_v[pl.ds(o + j0, _L)]
            i16 = ii_v[pl.ds(o + j0, _L)]
            raw = (tot + plsc.load_gather(buv, [u16])
                   + plsc.load_gather(bbv, [i16]) + bx16)
            out_v[pl.ds(o + j0, _L)] = 4.0 / (1.0 + jnp.exp(-raw)) + 1.0
            return _g
        lax.fori_loop(0, _FCK // _L, _group, None)
        return _
    lax.fori_loop(0, _BPT // _FCK, _chunk, None)

    pltpu.sync_copy(out_v, r_out.at[pl.ds(base, _BPT)])


def kernel(user_indices, item_indices, S_u, S_b, edge_indices_u,
           edge_indices_b, W1_u, a_self_u, a_nb_u, omega_u, W1_b, a_self_b,
           a_nb_b, omega_b, W_u_2, W_us_2, b_us_2, W_b_2, W_bs_2, b_bs_2,
           W_u_3, W_b_3, H_u_4, H_b_4, b_u_x, b_b_x, b_x):
    S = jnp.stack([S_u, S_b])
    W1 = jnp.stack([W1_u, W1_b])
    A = jnp.stack([jnp.concatenate([a_self_u, a_nb_u], axis=0),
                   jnp.concatenate([a_self_b, a_nb_b], axis=0)])
    H1, asan = _tc1(S, W1, A)

    edges = jnp.concatenate(
        [edge_indices_u, edge_indices_b]).astype(jnp.int32)
    srcf = edges[:, 0, :].reshape(-1)
    dstf = edges[:, 1, :].reshape(-1)
    dst5 = dstf.reshape(2 * _NG, _NSUB, _NBIG, _NSC, _SUB)
    h1cat = H1.reshape(2 * _N, _D)
    om = jnp.broadcast_to(
        jnp.concatenate([omega_u, omega_b]).astype(jnp.float32)[:, None],
        (2 * _NG, _L))
    H2a, H2b = _sc_main(srcf, dstf, dst5, h1cat, asan.reshape(-1), om)

    W2 = jnp.stack([W_u_2, W_b_2])
    Ws2 = jnp.stack([W_us_2, W_bs_2])
    b2 = jnp.stack([b_us_2, b_bs_2])[:, None, :]
    W3 = jnp.stack([W_u_3, W_b_3])
    H4 = jnp.stack([H_u_4, H_b_4])
    U = _tc2(H2a, H2b, S, W2, Ws2, b2, W3, H4)

    bub = jnp.concatenate([b_u_x[:, 0], b_b_x[:, 0]]).astype(jnp.float32)
    bx16 = jnp.full((_L,), b_x[0], jnp.float32)
    return _sc_final(U[0], U[1], user_indices.astype(jnp.int32),
                     item_indices.astype(jnp.int32), bub, bx16)
